# baseline (speedup 1.0000x reference)
import functools

import jax
import jax.numpy as jnp
from jax import lax
from jax.experimental import pallas as pl
from jax.experimental.pallas import tpu as pltpu
from jax.experimental.pallas import tpu_sc as plsc

_VOCAB = 1000000
_DIM = 64
_NC, _NS = 2, 16
_NW = 32
_NBLK_FULL = _VOCAB // 128
_K_FULL = _NBLK_FULL // _NW
_NBLK_EXTRA = _NBLK_FULL - _K_FULL * _NW
_TAIL_ROWS = _VOCAB - _NBLK_FULL * 128

_mesh = plsc.VectorSubcoreMesh(
    core_axis_name="c", subcore_axis_name="s",
    num_cores=_NC, num_subcores=_NS,
)


@functools.partial(
    pl.kernel,
    out_type=jax.ShapeDtypeStruct((_VOCAB, 128), jnp.float32),
    mesh=_mesh,
    scratch_types=[
        [pltpu.VMEM((_DIM, 128), jnp.float32) for _ in range(2)],
        [pltpu.VMEM((128, 128), jnp.float32) for _ in range(2)],
        [pltpu.SemaphoreType.DMA for _ in range(2)],
        [pltpu.SemaphoreType.DMA for _ in range(2)],
    ],
    compiler_params=pltpu.CompilerParams(use_tc_tiling_on_sc=True, needs_layout_passes=False),
)
def _format_kernel(tableT_hbm, tail_hbm, tpad_hbm, slab, tbuf, rsem, wsem):
    wid = lax.axis_index("s") * _NC + lax.axis_index("c")
    iota16 = lax.iota(jnp.int32, 16)

    def _block_of(k):
        return (wid + k * _NW) * 128

    def _transpose(si, ti):
        @pl.loop(0, 128)
        def _col(j):
            colv = jnp.full((16,), j, jnp.int32)
            for k in range(_DIM // 16):
                vals = plsc.load_gather(slab[si], [16 * k + iota16, colv])
                tbuf[ti][j, pl.ds(16 * k, 16)] = vals

    c0 = pl.multiple_of(_block_of(0), 128)
    pltpu.async_copy(tableT_hbm.at[:, pl.ds(c0, 128)], slab[0], rsem[0])

    @pl.loop(0, _K_FULL, step=2)
    def _blk(k0):
        for p in range(2):
            k = k0 + p
            si = p
            pltpu.make_async_copy(
                tableT_hbm.at[:, pl.ds(0, 128)], slab[si], rsem[si]).wait()

            @pl.when(k + 1 < _K_FULL)
            def _pref():
                cn = pl.multiple_of(_block_of(k + 1), 128)
                pltpu.async_copy(
                    tableT_hbm.at[:, pl.ds(cn, 128)], slab[1 - si],
                    rsem[1 - si])

            @pl.when(k >= 2)
            def _reuse():
                pltpu.make_async_copy(
                    tbuf[si], tpad_hbm.at[pl.ds(0, 128)],
                    wsem[si]).wait()

            _transpose(si, si)
            r0 = pl.multiple_of(_block_of(k), 128)
            pltpu.async_copy(
                tbuf[si], tpad_hbm.at[pl.ds(r0, 128)],
                wsem[si])

    for c in range(2):
        pltpu.make_async_copy(
            tbuf[c], tpad_hbm.at[pl.ds(0, 128)],
            wsem[c]).wait()

    @pl.when(wid < _NBLK_EXTRA)
    def _extra():
        c0x = pl.multiple_of((_K_FULL * _NW + wid) * 128, 128)
        pltpu.sync_copy(tableT_hbm.at[:, pl.ds(c0x, 128)], slab[0])
        _transpose(0, 0)
        pltpu.sync_copy(tbuf[0], tpad_hbm.at[pl.ds(c0x, 128)])

    @pl.when(wid == _NBLK_EXTRA)
    def _tail():
        t0 = pl.multiple_of(_NBLK_FULL * 128, 128)
        pltpu.sync_copy(tail_hbm, tpad_hbm.at[pl.ds(t0, _TAIL_ROWS)])


def _format_table(table):
    tail = jnp.pad(table[_NBLK_FULL * 128:], ((0, 0), (0, 64)))
    return _format_kernel(table.T, tail)


_B = 4096 * 200
_B_PER_W = _B // _NW
_CHUNK = 128
_N_CHUNKS = _B_PER_W // _CHUNK
_NBUF = 4


@functools.partial(
    pl.kernel,
    out_type=jax.ShapeDtypeStruct((_B, _DIM), jnp.float32),
    mesh=_mesh,
    scratch_types=[
        pltpu.VMEM((2 * _NBUF, _CHUNK), jnp.int32),
        [pltpu.VMEM((_CHUNK, 128), jnp.float32) for _ in range(_NBUF)],
        [pltpu.VMEM((_CHUNK, _DIM), jnp.float32) for _ in range(2)],
        [pltpu.SemaphoreType.DMA for _ in range(_NBUF)],
        [pltpu.SemaphoreType.DMA for _ in range(2)],
    ],
    compiler_params=pltpu.CompilerParams(use_tc_tiling_on_sc=True),
)
def _gather_kernel(idx_hbm, tpad_hbm, out_hbm, idx_v, rows, rows64,
                   gsem, wsem):
    wid = lax.axis_index("s") * _NC + lax.axis_index("c")
    chunk0 = wid * _N_CHUNKS
    base = wid * _B_PER_W

    pltpu.sync_copy(idx_hbm.at[pl.ds(chunk0, _NBUF)],
                    idx_v.at[pl.ds(0, _NBUF)])
    for b in range(_NBUF):
        pltpu.async_copy(tpad_hbm.at[idx_v.at[b]], rows[b], gsem[b])

    @pl.loop(0, _N_CHUNKS, step=_NBUF)
    def _slab(g0):
        for b in range(_NBUF):
            c = b % 2
            pltpu.make_async_copy(
                tpad_hbm.at[idx_v.at[b]], rows[b], gsem[b]).wait()

            @pl.when(g0 + b >= 2)
            def _reuse():
                pltpu.make_async_copy(
                    rows64[c], out_hbm.at[pl.ds(0, _CHUNK)],
                    wsem[c]).wait()

            @pl.loop(0, _CHUNK)
            def _row(i):
                for k in range(_DIM // 16):
                    rows64[c][i, pl.ds(16 * k, 16)] = (
                        rows[b][i, pl.ds(16 * k, 16)])

            pltpu.async_copy(
                rows64[c],
                out_hbm.at[pl.ds(base + (g0 + b) * _CHUNK, _CHUNK)],
                wsem[c])

        @pl.when(g0 + _NBUF < _N_CHUNKS)
        def _next():
            pltpu.sync_copy(
                idx_hbm.at[pl.ds(chunk0 + g0 + _NBUF, _NBUF)],
                idx_v.at[pl.ds(0, _NBUF)])
            for b in range(_NBUF):
                pltpu.async_copy(tpad_hbm.at[idx_v.at[b]], rows[b], gsem[b])

    for c in range(2):
        pltpu.make_async_copy(
            rows64[c], out_hbm.at[pl.ds(0, _CHUNK)], wsem[c]).wait()


def kernel(inputs, table):
    tpad = _format_table(table)
    idx = inputs.reshape(_B // _CHUNK, _CHUNK)
    out = _gather_kernel(idx, tpad)
    return out.reshape(4096, 200, _DIM)


# speedup vs baseline: 1.5911x; 1.5911x over previous
import functools

import jax
import jax.numpy as jnp
from jax import lax
from jax.experimental import pallas as pl
from jax.experimental.pallas import tpu as pltpu
from jax.experimental.pallas import tpu_sc as plsc

_VOCAB = 1000000
_DIM = 64
_B = 4096 * 200
_NC, _NS = 2, 16
_NW = 32
_B_PER_W = _B // _NW
_CHUNK = 128
_N_CHUNKS = _B_PER_W // _CHUNK
_NBUF = 4

_mesh = plsc.VectorSubcoreMesh(
    core_axis_name="c", subcore_axis_name="s",
    num_cores=_NC, num_subcores=_NS,
)


@functools.partial(
    pl.kernel,
    out_type=jax.ShapeDtypeStruct((_B, _DIM), jnp.float32),
    mesh=_mesh,
    scratch_types=[
        pltpu.VMEM((2 * _NBUF, _CHUNK), jnp.int32),
        [pltpu.VMEM((_CHUNK, 128), jnp.float32) for _ in range(_NBUF)],
        [pltpu.VMEM((_CHUNK, _DIM), jnp.float32) for _ in range(2)],
        [pltpu.SemaphoreType.DMA for _ in range(_NBUF)],
        [pltpu.SemaphoreType.DMA for _ in range(2)],
    ],
    compiler_params=pltpu.CompilerParams(use_tc_tiling_on_sc=True),
)
def _gather_kernel(idx_hbm, tpad_hbm, out_hbm, idx_v, rows, rows64,
                   gsem, wsem):
    wid = lax.axis_index("s") * _NC + lax.axis_index("c")
    chunk0 = wid * _N_CHUNKS
    base = wid * _B_PER_W

    pltpu.sync_copy(idx_hbm.at[pl.ds(chunk0, _NBUF)],
                    idx_v.at[pl.ds(0, _NBUF)])
    for b in range(_NBUF):
        pltpu.async_copy(tpad_hbm.at[idx_v.at[b]], rows[b], gsem[b])

    @pl.loop(0, _N_CHUNKS, step=_NBUF)
    def _slab(g0):
        for b in range(_NBUF):
            c = b % 2
            pltpu.make_async_copy(
                tpad_hbm.at[idx_v.at[b]], rows[b], gsem[b]).wait()

            @pl.when(g0 + b >= 2)
            def _reuse():
                pltpu.make_async_copy(
                    rows64[c], out_hbm.at[pl.ds(0, _CHUNK)],
                    wsem[c]).wait()

            @pl.loop(0, _CHUNK, unroll=8)
            def _row(i):
                for k in range(_DIM // 16):
                    rows64[c][i, pl.ds(16 * k, 16)] = (
                        rows[b][i, pl.ds(16 * k, 16)])

            pltpu.async_copy(
                rows64[c],
                out_hbm.at[pl.ds(base + (g0 + b) * _CHUNK, _CHUNK)],
                wsem[c])

        @pl.when(g0 + _NBUF < _N_CHUNKS)
        def _next():
            pltpu.sync_copy(
                idx_hbm.at[pl.ds(chunk0 + g0 + _NBUF, _NBUF)],
                idx_v.at[pl.ds(0, _NBUF)])
            for b in range(_NBUF):
                pltpu.async_copy(tpad_hbm.at[idx_v.at[b]], rows[b], gsem[b])

    for c in range(2):
        pltpu.make_async_copy(
            rows64[c], out_hbm.at[pl.ds(0, _CHUNK)], wsem[c]).wait()


def kernel(inputs, table):
    tpad = jnp.pad(table, ((0, 0), (0, 64)))
    idx = inputs.reshape(_B // _CHUNK, _CHUNK)
    out = _gather_kernel(idx, tpad)
    return out.reshape(4096, 200, 64)
